# baseline (speedup 1.0000x reference)
import functools

import jax
import jax.numpy as jnp
from jax.experimental import pallas as pl
from jax.experimental.pallas import tpu as pltpu

_EPS = 1e-5
_LEAK = 0.1


def _conv3x3_stats_kernel(x_ref, w_ref, y_ref, s_ref, *, H, W, IC, OCt):
    o = pl.program_id(0)
    b = pl.program_id(1)

    acc = jnp.zeros((H * W, OCt), jnp.float32)
    for k in range(9):
        dy, dx = k // 3, k % 3
        xs = x_ref[0, dy:dy + H, dx:dx + W, :].reshape(H * W, IC)
        acc = acc + jnp.dot(xs, w_ref[k], preferred_element_type=jnp.float32)
    y_ref[...] = acc
    s1 = jnp.sum(acc, axis=0)
    s2 = jnp.sum(acc * acc, axis=0)
    sv = jnp.stack([s1, s2], axis=0)

    @pl.when(b == 0)
    def _():
        s_ref[:, pl.ds(o * OCt, OCt)] = sv

    @pl.when(b != 0)
    def _():
        s_ref[:, pl.ds(o * OCt, OCt)] = s_ref[:, pl.ds(o * OCt, OCt)] + sv


def _bn_leaky_mm_kernel(y_ref, s_ref, g_ref, bb_ref, w2_ref, b2_ref, o_ref,
                        *, n):
    mean = s_ref[0:1, :] * (1.0 / n)
    var = s_ref[1:2, :] * (1.0 / n) - mean * mean
    scale = g_ref[...] * jax.lax.rsqrt(var + _EPS)
    shift = bb_ref[...] - mean * scale
    h = y_ref[...] * scale + shift
    h = jnp.maximum(h, _LEAK * h)
    mm = jax.lax.dot_general(h, w2_ref[...], (((1,), (1,)), ((), ())),
                             preferred_element_type=jnp.float32)
    o_ref[...] = mm + b2_ref[...]


def _head_scale(x, p, *, oct_conv, rows_tile):
    B, IC, H, W = x.shape
    OC = 1024
    xh = jnp.pad(jnp.transpose(x, (0, 2, 3, 1)),
                 ((0, 0), (1, 1), (1, 1), (0, 0)))
    w1r = jnp.transpose(p['w1'], (2, 3, 1, 0)).reshape(9, IC, OC)

    hw = H * W
    n = B * hw
    ocn = OC // oct_conv
    y, s = pl.pallas_call(
        functools.partial(_conv3x3_stats_kernel, H=H, W=W, IC=IC,
                          OCt=oct_conv),
        grid=(ocn, B),
        in_specs=[
            pl.BlockSpec((1, H + 2, W + 2, IC), lambda o, b: (b, 0, 0, 0)),
            pl.BlockSpec((9, IC, oct_conv), lambda o, b: (0, 0, o)),
        ],
        out_specs=[
            pl.BlockSpec((hw, oct_conv), lambda o, b: (b, o)),
            pl.BlockSpec((2, OC), lambda o, b: (0, 0)),
        ],
        out_shape=[
            jax.ShapeDtypeStruct((n, OC), jnp.float32),
            jax.ShapeDtypeStruct((2, OC), jnp.float32),
        ],
    )(xh, w1r)

    oc2 = p['w2'].shape[0]
    w2 = p['w2'].reshape(oc2, OC)
    b2 = p['b2'].reshape(1, oc2)
    g = p['g'].reshape(1, OC)
    bb = p['b'].reshape(1, OC)

    tr = min(rows_tile, n)
    out = pl.pallas_call(
        functools.partial(_bn_leaky_mm_kernel, n=n),
        grid=(n // tr,),
        in_specs=[
            pl.BlockSpec((tr, OC), lambda r: (r, 0)),
            pl.BlockSpec((2, OC), lambda r: (0, 0)),
            pl.BlockSpec((1, OC), lambda r: (0, 0)),
            pl.BlockSpec((1, OC), lambda r: (0, 0)),
            pl.BlockSpec((oc2, OC), lambda r: (0, 0)),
            pl.BlockSpec((1, oc2), lambda r: (0, 0)),
        ],
        out_specs=pl.BlockSpec((tr, oc2), lambda r: (r, 0)),
        out_shape=jax.ShapeDtypeStruct((n, oc2), jnp.float32),
    )(y, s, g, bb, w2, b2)

    return out.reshape(B, H, W, oc2)


@jax.jit
def kernel(feat0, feat1, feat2, params):
    out2 = _head_scale(feat2, params[2], oct_conv=512, rows_tile=1024)
    out1 = _head_scale(feat1, params[1], oct_conv=512, rows_tile=2048)
    out0 = _head_scale(feat0, params[0], oct_conv=512, rows_tile=2048)
    return (out0, out1, out2)


# speedup vs baseline: 1.1516x; 1.1516x over previous
import functools

import jax
import jax.numpy as jnp
from jax.experimental import pallas as pl
from jax.experimental.pallas import tpu as pltpu

_EPS = 1e-5
_LEAK = 0.1


def _conv3x3_stats_kernel(x_ref, w_ref, y_ref, s_ref, xp_ref, *, H, W, IC,
                          OCt):
    o = pl.program_id(0)
    b = pl.program_id(1)

    @pl.when(jnp.logical_and(b == 0, o == 0))
    def _():
        xp_ref[...] = jnp.zeros_like(xp_ref)

    xp_ref[1:H + 1, 1:W + 1, :] = x_ref[0]

    acc = jnp.zeros((H * W, OCt), jnp.float32)
    for k in range(9):
        dy, dx = k // 3, k % 3
        xs = xp_ref[dy:dy + H, dx:dx + W, :].reshape(H * W, IC)
        acc = acc + jnp.dot(xs, w_ref[k], preferred_element_type=jnp.float32)
    y_ref[...] = acc
    s1 = jnp.sum(acc, axis=0)
    s2 = jnp.sum(acc * acc, axis=0)
    sv = jnp.stack([s1, s2], axis=0)

    @pl.when(b == 0)
    def _():
        s_ref[:, pl.ds(o * OCt, OCt)] = sv

    @pl.when(b != 0)
    def _():
        s_ref[:, pl.ds(o * OCt, OCt)] = s_ref[:, pl.ds(o * OCt, OCt)] + sv


def _bn_leaky_mm_kernel(y_ref, s_ref, g_ref, bb_ref, w2_ref, b2_ref, o_ref,
                        *, n):
    mean = s_ref[0:1, :] * (1.0 / n)
    var = s_ref[1:2, :] * (1.0 / n) - mean * mean
    scale = g_ref[...] * jax.lax.rsqrt(var + _EPS)
    shift = bb_ref[...] - mean * scale
    h = y_ref[...] * scale + shift
    h = jnp.maximum(h, _LEAK * h)
    mm = jax.lax.dot_general(h, w2_ref[...], (((1,), (1,)), ((), ())),
                             preferred_element_type=jnp.float32)
    o_ref[...] = mm + b2_ref[...]


def _head_scale(x, p, *, oct_conv, rows_tile):
    B, IC, H, W = x.shape
    OC = 1024
    xh = jnp.transpose(x, (0, 2, 3, 1))
    w1r = jnp.transpose(p['w1'], (2, 3, 1, 0)).reshape(9, IC, OC)

    hw = H * W
    n = B * hw
    ocn = OC // oct_conv
    y, s = pl.pallas_call(
        functools.partial(_conv3x3_stats_kernel, H=H, W=W, IC=IC,
                          OCt=oct_conv),
        grid=(ocn, B),
        in_specs=[
            pl.BlockSpec((1, H, W, IC), lambda o, b: (b, 0, 0, 0)),
            pl.BlockSpec((9, IC, oct_conv), lambda o, b: (0, 0, o)),
        ],
        out_specs=[
            pl.BlockSpec((hw, oct_conv), lambda o, b: (b, o)),
            pl.BlockSpec((2, OC), lambda o, b: (0, 0)),
        ],
        out_shape=[
            jax.ShapeDtypeStruct((n, OC), jnp.float32),
            jax.ShapeDtypeStruct((2, OC), jnp.float32),
        ],
        scratch_shapes=[pltpu.VMEM((H + 2, W + 2, IC), jnp.float32)],
    )(xh, w1r)

    oc2 = p['w2'].shape[0]
    w2 = p['w2'].reshape(oc2, OC)
    b2 = p['b2'].reshape(1, oc2)
    g = p['g'].reshape(1, OC)
    bb = p['b'].reshape(1, OC)

    tr = min(rows_tile, n)
    out = pl.pallas_call(
        functools.partial(_bn_leaky_mm_kernel, n=n),
        grid=(n // tr,),
        in_specs=[
            pl.BlockSpec((tr, OC), lambda r: (r, 0)),
            pl.BlockSpec((2, OC), lambda r: (0, 0)),
            pl.BlockSpec((1, OC), lambda r: (0, 0)),
            pl.BlockSpec((1, OC), lambda r: (0, 0)),
            pl.BlockSpec((oc2, OC), lambda r: (0, 0)),
            pl.BlockSpec((1, oc2), lambda r: (0, 0)),
        ],
        out_specs=pl.BlockSpec((tr, oc2), lambda r: (r, 0)),
        out_shape=jax.ShapeDtypeStruct((n, oc2), jnp.float32),
    )(y, s, g, bb, w2, b2)

    return out.reshape(B, H, W, oc2)


@jax.jit
def kernel(feat0, feat1, feat2, params):
    out2 = _head_scale(feat2, params[2], oct_conv=512, rows_tile=1024)
    out1 = _head_scale(feat1, params[1], oct_conv=512, rows_tile=4096)
    out0 = _head_scale(feat0, params[0], oct_conv=512, rows_tile=4096)
    return (out0, out1, out2)


# speedup vs baseline: 1.1614x; 1.0085x over previous
import functools

import jax
import jax.numpy as jnp
from jax.experimental import pallas as pl
from jax.experimental.pallas import tpu as pltpu

_EPS = 1e-5
_LEAK = 0.1


def _conv3x3_stats_kernel(x_ref, w_ref, y_ref, s_ref, xp_ref, *, H, W, IC,
                          OCt):
    o = pl.program_id(0)
    b = pl.program_id(1)

    @pl.when(jnp.logical_and(b == 0, o == 0))
    def _():
        xp_ref[...] = jnp.zeros_like(xp_ref)

    xp_ref[1:H + 1, 1:W + 1, :] = x_ref[0]

    acc = jnp.zeros((H * W, OCt), jnp.float32)
    for k in range(9):
        dy, dx = k // 3, k % 3
        xs = xp_ref[dy:dy + H, dx:dx + W, :].reshape(H * W, IC)
        acc = acc + jnp.dot(xs, w_ref[k], preferred_element_type=jnp.float32)
    y_ref[...] = acc
    s1 = jnp.sum(acc, axis=0)
    s2 = jnp.sum(acc * acc, axis=0)
    sv = jnp.stack([s1, s2], axis=0)

    @pl.when(b == 0)
    def _():
        s_ref[:, pl.ds(o * OCt, OCt)] = sv

    @pl.when(b != 0)
    def _():
        s_ref[:, pl.ds(o * OCt, OCt)] = s_ref[:, pl.ds(o * OCt, OCt)] + sv


def _bn_leaky_mm_kernel(y_ref, s_ref, g_ref, bb_ref, w2_ref, b2_ref, o_ref,
                        *, n):
    mean = s_ref[0:1, :] * (1.0 / n)
    var = s_ref[1:2, :] * (1.0 / n) - mean * mean
    scale = g_ref[...] * jax.lax.rsqrt(var + _EPS)
    shift = bb_ref[...] - mean * scale
    h = y_ref[...] * scale + shift
    h = jnp.maximum(h, _LEAK * h)
    mm = jax.lax.dot_general(h, w2_ref[...], (((1,), (1,)), ((), ())),
                             preferred_element_type=jnp.float32)
    o_ref[...] = mm + b2_ref[...]


def _head_scale(x, p, *, oct_conv, rows_tile):
    B, IC, H, W = x.shape
    OC = 1024
    xh = jnp.transpose(x, (0, 2, 3, 1))
    w1r = jnp.transpose(p['w1'], (2, 3, 1, 0)).reshape(9, IC, OC)

    hw = H * W
    n = B * hw
    ocn = OC // oct_conv
    y, s = pl.pallas_call(
        functools.partial(_conv3x3_stats_kernel, H=H, W=W, IC=IC,
                          OCt=oct_conv),
        grid=(ocn, B),
        in_specs=[
            pl.BlockSpec((1, H, W, IC), lambda o, b: (b, 0, 0, 0)),
            pl.BlockSpec((9, IC, oct_conv), lambda o, b: (0, 0, o)),
        ],
        out_specs=[
            pl.BlockSpec((hw, oct_conv), lambda o, b: (b, o)),
            pl.BlockSpec((2, OC), lambda o, b: (0, 0)),
        ],
        out_shape=[
            jax.ShapeDtypeStruct((n, OC), jnp.float32),
            jax.ShapeDtypeStruct((2, OC), jnp.float32),
        ],
        scratch_shapes=[pltpu.VMEM((H + 2, W + 2, IC), jnp.float32)],
    )(xh, w1r)

    oc2 = p['w2'].shape[0]
    w2 = p['w2'].reshape(oc2, OC)
    b2 = p['b2'].reshape(1, oc2)
    g = p['g'].reshape(1, OC)
    bb = p['b'].reshape(1, OC)

    tr = min(rows_tile, n)
    out = pl.pallas_call(
        functools.partial(_bn_leaky_mm_kernel, n=n),
        grid=(n // tr,),
        in_specs=[
            pl.BlockSpec((tr, OC), lambda r: (r, 0)),
            pl.BlockSpec((2, OC), lambda r: (0, 0)),
            pl.BlockSpec((1, OC), lambda r: (0, 0)),
            pl.BlockSpec((1, OC), lambda r: (0, 0)),
            pl.BlockSpec((oc2, OC), lambda r: (0, 0)),
            pl.BlockSpec((1, oc2), lambda r: (0, 0)),
        ],
        out_specs=pl.BlockSpec((tr, oc2), lambda r: (r, 0)),
        out_shape=jax.ShapeDtypeStruct((n, oc2), jnp.float32),
    )(y, s, g, bb, w2, b2)

    return out.reshape(B, H, W, oc2)


@jax.jit
def kernel(feat0, feat1, feat2, params):
    out2 = _head_scale(feat2, params[2], oct_conv=512, rows_tile=1024)
    out1 = _head_scale(feat1, params[1], oct_conv=512, rows_tile=2048)
    out0 = _head_scale(feat0, params[0], oct_conv=512, rows_tile=2048)
    return (out0, out1, out2)
